# Initial kernel scaffold; baseline (speedup 1.0000x reference)
import jax
import jax.numpy as jnp
from jax.experimental import pallas as pl


def kernel(x, edge_index, batch, W0, b0, W1, b1, W2, b2, Wl, bl):
    raise NotImplementedError("write your pallas kernel here")


# speedup vs baseline: 9.6800x; 9.6800x over previous
import functools

import jax
import jax.numpy as jnp
from jax import lax
from jax.experimental import pallas as pl
from jax.experimental.pallas import tpu as pltpu
from jax.experimental.pallas import tpu_sc as plsc

N = 100000
E = 1600000
H = 64
G = 128
Q = 4
NPT = 6272
ND = 16 * NPT
DUMP = N
EPT = 102400
KCH = 10
BLKE = KCH * 128
EP = 16 * EPT
EPW = EP // 32
NH = 100352
BN = 2000
NB = N // BN

_MESH = dict(core_axis_name="c", subcore_axis_name="s", num_cores=2,
             num_subcores=16)


def _sc_deg(dst2d):
    mesh = plsc.VectorSubcoreMesh(**_MESH)

    @functools.partial(
        pl.kernel,
        out_type=jax.ShapeDtypeStruct((32, NH), jnp.float32),
        mesh=mesh,
        compiler_params=pltpu.CompilerParams(needs_layout_passes=False),
        scratch_types=[
            pltpu.VMEM((16, 128), jnp.int32),
            pltpu.VMEM((NH,), jnp.float32),
        ],
    )
    def k(dst_h, out_h, dst_v, hist):
        c = lax.axis_index("c")
        s = lax.axis_index("s")
        w = s * 2 + c

        @pl.loop(0, NH // 16)
        def _zero(i):
            hist[pl.ds(i * 16, 16)] = jnp.zeros((16,), jnp.float32)

        row0 = w * (EPW // 128)
        ones = jnp.ones((16,), jnp.float32)

        @pl.loop(0, EPW // 2048)
        def _blk(b):
            pltpu.sync_copy(dst_h.at[pl.ds(row0 + b * 16, 16)], dst_v)
            for kk in range(16):
                for ii in range(8):
                    idx = dst_v[kk, pl.ds(ii * 16, 16)]
                    plsc.addupdate_scatter(hist, [idx], ones)

        pltpu.sync_copy(hist.at[pl.ds(0, NH)], out_h.at[w])

    return k(dst2d)


def _sc_scatter(yflat, srcq, dst2d):
    mesh = plsc.VectorSubcoreMesh(**_MESH)

    @functools.partial(
        pl.kernel,
        out_type=jax.ShapeDtypeStruct((Q * ND, 16), jnp.float32),
        mesh=mesh,
        compiler_params=pltpu.CompilerParams(needs_layout_passes=False,
                                             use_tc_tiling_on_sc=False),
        scratch_types=[
            pltpu.VMEM((KCH, 128), jnp.int32),
            pltpu.VMEM((KCH, 128), jnp.int32),
            pltpu.VMEM((BLKE, 16), jnp.float32),
            pltpu.VMEM_SHARED((ND, 16), jnp.float32),
            pltpu.SemaphoreType.DMA,
        ],
    )
    def k(y_h, srcq_h, dst_h, out_h, src_v, dst_v, rows_v, acc, sem):
        c = lax.axis_index("c")
        s = lax.axis_index("s")

        row0 = s * (EPT // 128)
        for j in range(2):
            q = 2 * c + j

            @pl.loop(0, BLKE)
            def _zb(i):
                rows_v[i, :] = jnp.zeros((16,), jnp.float32)

            for b in range(4):
                pltpu.sync_copy(rows_v,
                                acc.at[pl.ds(s * NPT + b * BLKE, BLKE)])
            pltpu.sync_copy(rows_v.at[pl.ds(0, NPT - 4 * BLKE)],
                            acc.at[pl.ds(s * NPT + 4 * BLKE, NPT - 4 * BLKE)])

            plsc.subcore_barrier()

            @pl.loop(0, EPT // BLKE)
            def _blk(b):
                r = row0 + b * KCH
                pltpu.sync_copy(srcq_h.at[q, pl.ds(r, KCH)], src_v)
                pltpu.sync_copy(dst_h.at[pl.ds(r, KCH)], dst_v)
                copies = []
                for kk in range(KCH):
                    copies.append(pltpu.async_copy(
                        y_h.at[src_v.at[kk]],
                        rows_v.at[pl.ds(kk * 128, 128)], sem))
                for cp in copies:
                    cp.wait()
                for kk in range(KCH):
                    pltpu.sync_copy(rows_v.at[pl.ds(kk * 128, 128)],
                                    acc.at[dst_v.at[kk]], add=True)

            plsc.subcore_barrier()
            pltpu.sync_copy(acc.at[pl.ds(s * NPT, NPT)],
                            out_h.at[pl.ds(q * ND + s * NPT, NPT)])
            plsc.subcore_barrier()

    return k(yflat, srcq, dst2d)


def _tc_dinv(parts):
    bn = 2048

    def body(p_ref, o_ref):
        sm = jnp.sum(p_ref[...], axis=0, keepdims=True) + 1.0
        o_ref[...] = lax.rsqrt(sm)

    return pl.pallas_call(
        body,
        grid=(pl.cdiv(N, bn),),
        in_specs=[pl.BlockSpec((32, bn), lambda i: (0, i))],
        out_specs=pl.BlockSpec((1, bn), lambda i: (0, i)),
        out_shape=jax.ShapeDtypeStruct((1, N), jnp.float32),
    )(parts)


def _tc_first(xp, Wp, dinv_col):
    def body(x_ref, w_ref, d_ref, y_ref):
        h = jnp.dot(x_ref[...], w_ref[...],
                    preferred_element_type=jnp.float32) * d_ref[...]
        for q in range(Q):
            y_ref[q] = h[:, q * 16:(q + 1) * 16]

    return pl.pallas_call(
        body,
        grid=(NB,),
        in_specs=[
            pl.BlockSpec((BN, 32), lambda i: (i, 0)),
            pl.BlockSpec((32, 64), lambda i: (0, 0)),
            pl.BlockSpec((BN, 1), lambda i: (i, 0)),
        ],
        out_specs=pl.BlockSpec((Q, BN, 16), lambda i: (0, i, 0)),
        out_shape=jax.ShapeDtypeStruct((Q, N, 16), jnp.float32),
    )(xp, Wp, dinv_col)


def _tc_mid(S, y, dinv_col, b_row, Wn):
    def body(s_ref, y_ref, d_ref, b_ref, w_ref, o_ref):
        sf = jnp.concatenate([s_ref[q] for q in range(Q)], axis=1)
        yf = jnp.concatenate([y_ref[q] for q in range(Q)], axis=1)
        e = d_ref[...] * (sf + yf) + b_ref[...]
        h = jnp.maximum(e, 0.0)
        hw = jnp.dot(h, w_ref[...],
                     preferred_element_type=jnp.float32) * d_ref[...]
        for q in range(Q):
            o_ref[q] = hw[:, q * 16:(q + 1) * 16]

    return pl.pallas_call(
        body,
        grid=(NB,),
        in_specs=[
            pl.BlockSpec((Q, BN, 16), lambda i: (0, i, 0)),
            pl.BlockSpec((Q, BN, 16), lambda i: (0, i, 0)),
            pl.BlockSpec((BN, 1), lambda i: (i, 0)),
            pl.BlockSpec((1, 64), lambda i: (0, 0)),
            pl.BlockSpec((64, 64), lambda i: (0, 0)),
        ],
        out_specs=pl.BlockSpec((Q, BN, 16), lambda i: (0, i, 0)),
        out_shape=jax.ShapeDtypeStruct((Q, N, 16), jnp.float32),
    )(S, y, dinv_col, b_row, Wn)


def _tc_final(S, y, dinv_col, b_row, Wl, bl2, batch_col):
    def body(s_ref, y_ref, d_ref, b_ref, wl_ref, bl_ref, bat_ref, o_ref, acc):
        i = pl.program_id(0)
        sf = jnp.concatenate([s_ref[q] for q in range(Q)], axis=1)
        yf = jnp.concatenate([y_ref[q] for q in range(Q)], axis=1)
        e = d_ref[...] * (sf + yf) + b_ref[...]
        v = jnp.dot(e, wl_ref[...], preferred_element_type=jnp.float32)
        onehot = (bat_ref[...] == lax.broadcasted_iota(
            jnp.int32, (BN, G), 1)).astype(jnp.float32)
        vc = jnp.concatenate([v, jnp.ones((BN, 1), jnp.float32)], axis=1)
        part = lax.dot_general(vc, onehot, (((0,), (0,)), ((), ())),
                               preferred_element_type=jnp.float32)

        @pl.when(i == 0)
        def _():
            acc[...] = jnp.zeros((2, G), jnp.float32)

        acc[...] += part

        @pl.when(i == NB - 1)
        def _():
            res = acc[0, :] / jnp.maximum(acc[1, :], 1.0) + bl_ref[0, 0]
            o_ref[...] = res[None, :]

    return pl.pallas_call(
        body,
        grid=(NB,),
        in_specs=[
            pl.BlockSpec((Q, BN, 16), lambda i: (0, i, 0)),
            pl.BlockSpec((Q, BN, 16), lambda i: (0, i, 0)),
            pl.BlockSpec((BN, 1), lambda i: (i, 0)),
            pl.BlockSpec((1, 64), lambda i: (0, 0)),
            pl.BlockSpec((64, 1), lambda i: (0, 0)),
            pl.BlockSpec((1, 1), lambda i: (0, 0)),
            pl.BlockSpec((BN, 1), lambda i: (i, 0)),
        ],
        out_specs=pl.BlockSpec((1, G), lambda i: (0, 0)),
        out_shape=jax.ShapeDtypeStruct((1, G), jnp.float32),
        scratch_shapes=[pltpu.VMEM((2, G), jnp.float32)],
    )(S, y, dinv_col, b_row, Wl, bl2, batch_col)


def kernel(x, edge_index, batch, W0, b0, W1, b1, W2, b2, Wl, bl):
    src = edge_index[0]
    dst = edge_index[1]
    pad = EP - E
    src_p = jnp.concatenate([src, jnp.zeros((pad,), jnp.int32)])
    dst_p = jnp.concatenate([dst, jnp.full((pad,), DUMP, jnp.int32)])
    srcq = (src_p[None, :]
            + (jnp.arange(Q, dtype=jnp.int32) * N)[:, None]).reshape(
                Q, EP // 128, 128)
    dst2d = dst_p.reshape(EP // 128, 128)

    xp = jnp.pad(x, ((0, 0), (0, 32 - x.shape[1])))
    W0p = jnp.pad(W0, ((0, 32 - W0.shape[0]), (0, 0)))

    parts = _sc_deg(dst2d)
    dinv_col = _tc_dinv(parts).reshape(N, 1)
    batch_col = batch.reshape(N, 1)

    y = _tc_first(xp, W0p, dinv_col)
    S = _sc_scatter(y.reshape(Q * N, 16), srcq, dst2d).reshape(Q, ND, 16)
    y = _tc_mid(S, y, dinv_col, b0.reshape(1, H), W1)
    S = _sc_scatter(y.reshape(Q * N, 16), srcq, dst2d).reshape(Q, ND, 16)
    y = _tc_mid(S, y, dinv_col, b1.reshape(1, H), W2)
    S = _sc_scatter(y.reshape(Q * N, 16), srcq, dst2d).reshape(Q, ND, 16)
    out = _tc_final(S, y, dinv_col, b2.reshape(1, H), Wl,
                    bl.reshape(1, 1), batch_col)
    return out.reshape(G, 1)


# speedup vs baseline: 10.7066x; 1.1060x over previous
import functools

import jax
import jax.numpy as jnp
from jax import lax
from jax.experimental import pallas as pl
from jax.experimental.pallas import tpu as pltpu
from jax.experimental.pallas import tpu_sc as plsc

N = 100000
E = 1600000
H = 64
G = 128
Q = 4
NPT = 6272
ND = 16 * NPT
DUMP = N
EPT = 102400
KCH = 5
BLKE = KCH * 128
EP = 16 * EPT
EPW = EP // 32
NH = 100352
BN = 2000
NB = N // BN

_MESH = dict(core_axis_name="c", subcore_axis_name="s", num_cores=2,
             num_subcores=16)


def _sc_deg(dst2d):
    mesh = plsc.VectorSubcoreMesh(**_MESH)

    @functools.partial(
        pl.kernel,
        out_type=jax.ShapeDtypeStruct((32, NH), jnp.float32),
        mesh=mesh,
        compiler_params=pltpu.CompilerParams(needs_layout_passes=False),
        scratch_types=[
            pltpu.VMEM((16, 128), jnp.int32),
            pltpu.VMEM((NH,), jnp.float32),
        ],
    )
    def k(dst_h, out_h, dst_v, hist):
        c = lax.axis_index("c")
        s = lax.axis_index("s")
        w = s * 2 + c

        @pl.loop(0, NH // 16)
        def _zero(i):
            hist[pl.ds(i * 16, 16)] = jnp.zeros((16,), jnp.float32)

        row0 = w * (EPW // 128)
        ones = jnp.ones((16,), jnp.float32)

        @pl.loop(0, EPW // 2048)
        def _blk(b):
            pltpu.sync_copy(dst_h.at[pl.ds(row0 + b * 16, 16)], dst_v)
            for kk in range(16):
                for ii in range(8):
                    idx = dst_v[kk, pl.ds(ii * 16, 16)]
                    plsc.addupdate_scatter(hist, [idx], ones)

        pltpu.sync_copy(hist.at[pl.ds(0, NH)], out_h.at[w])

    return k(dst2d)


def _sc_scatter(yflat, comb):
    mesh = plsc.VectorSubcoreMesh(**_MESH)

    @functools.partial(
        pl.kernel,
        out_type=jax.ShapeDtypeStruct((Q * ND, 16), jnp.float32),
        mesh=mesh,
        compiler_params=pltpu.CompilerParams(needs_layout_passes=False,
                                             use_tc_tiling_on_sc=False),
        scratch_types=[
            pltpu.VMEM((KCH, 2, 128), jnp.int32),
            pltpu.VMEM((KCH, 2, 128), jnp.int32),
            pltpu.VMEM((BLKE, 16), jnp.float32),
            pltpu.VMEM((BLKE, 16), jnp.float32),
            pltpu.VMEM_SHARED((ND, 16), jnp.float32),
            pltpu.SemaphoreType.DMA,
            pltpu.SemaphoreType.DMA,
        ],
    )
    def k(y_h, comb_h, out_h, idx_a, idx_b, rows_a, rows_b, acc,
          sem_g, sem_s):
        c = lax.axis_index("c")
        s = lax.axis_index("s")
        idx = (idx_a, idx_b)
        rows = (rows_a, rows_b)
        nblk = EPT // BLKE

        def fire_gathers(p, qq, bn):
            pltpu.sync_copy(
                comb_h.at[qq, pl.ds(s * (EPT // 128) + bn * KCH, KCH)],
                idx[p])
            for kk in range(KCH):
                pltpu.async_copy(y_h.at[idx[p].at[kk, 0]],
                                 rows[p].at[pl.ds(kk * 128, 128)], sem_g)

        def fire_scatters(p):
            for kk in range(KCH):
                pltpu.async_copy(rows[p].at[pl.ds(kk * 128, 128)],
                                 acc.at[idx[p].at[kk, 1]], sem_s, add=True)

        def drain(sem):
            pltpu.make_async_copy(y_h.at[pl.ds(0, BLKE)], rows_a, sem).wait()

        for j in range(2):
            q = 2 * c + j

            @pl.loop(0, BLKE)
            def _zb(i):
                rows_a[i, :] = jnp.zeros((16,), jnp.float32)

            nz = NPT // BLKE
            for b in range(nz):
                pltpu.sync_copy(rows_a,
                                acc.at[pl.ds(s * NPT + b * BLKE, BLKE)])
            if NPT % BLKE:
                pltpu.sync_copy(rows_a.at[pl.ds(0, NPT - nz * BLKE)],
                                acc.at[pl.ds(s * NPT + nz * BLKE,
                                             NPT - nz * BLKE)])

            plsc.subcore_barrier()

            fire_gathers(0, q, 0)

            @pl.loop(0, nblk // 2)
            def _blk(i):
                for p in range(2):
                    bn = 2 * i + p
                    drain(sem_g)
                    if p == 0:
                        @pl.when(bn > 0)
                        def _():
                            drain(sem_s)
                        fire_gathers(1, q, bn + 1)
                    else:
                        drain(sem_s)

                        @pl.when(bn + 1 < nblk)
                        def _():
                            fire_gathers(0, q, bn + 1)
                    fire_scatters(p)

            drain(sem_s)
            plsc.subcore_barrier()
            pltpu.sync_copy(acc.at[pl.ds(s * NPT, NPT)],
                            out_h.at[pl.ds(q * ND + s * NPT, NPT)])
            plsc.subcore_barrier()

    return k(yflat, comb)


def _tc_dinv(parts):
    bn = 2048

    def body(p_ref, o_ref):
        sm = jnp.sum(p_ref[...], axis=0, keepdims=True) + 1.0
        o_ref[...] = lax.rsqrt(sm)

    return pl.pallas_call(
        body,
        grid=(pl.cdiv(N, bn),),
        in_specs=[pl.BlockSpec((32, bn), lambda i: (0, i))],
        out_specs=pl.BlockSpec((1, bn), lambda i: (0, i)),
        out_shape=jax.ShapeDtypeStruct((1, N), jnp.float32),
    )(parts)


def _tc_first(xp, Wp, dinv_col):
    def body(x_ref, w_ref, d_ref, y_ref):
        h = jnp.dot(x_ref[...], w_ref[...],
                    preferred_element_type=jnp.float32) * d_ref[...]
        for q in range(Q):
            y_ref[q] = h[:, q * 16:(q + 1) * 16]

    return pl.pallas_call(
        body,
        grid=(NB,),
        in_specs=[
            pl.BlockSpec((BN, 32), lambda i: (i, 0)),
            pl.BlockSpec((32, 64), lambda i: (0, 0)),
            pl.BlockSpec((BN, 1), lambda i: (i, 0)),
        ],
        out_specs=pl.BlockSpec((Q, BN, 16), lambda i: (0, i, 0)),
        out_shape=jax.ShapeDtypeStruct((Q, N, 16), jnp.float32),
    )(xp, Wp, dinv_col)


def _tc_mid(S, y, dinv_col, b_row, Wn):
    def body(s_ref, y_ref, d_ref, b_ref, w_ref, o_ref):
        sf = jnp.concatenate([s_ref[q] for q in range(Q)], axis=1)
        yf = jnp.concatenate([y_ref[q] for q in range(Q)], axis=1)
        e = d_ref[...] * (sf + yf) + b_ref[...]
        h = jnp.maximum(e, 0.0)
        hw = jnp.dot(h, w_ref[...],
                     preferred_element_type=jnp.float32) * d_ref[...]
        for q in range(Q):
            o_ref[q] = hw[:, q * 16:(q + 1) * 16]

    return pl.pallas_call(
        body,
        grid=(NB,),
        in_specs=[
            pl.BlockSpec((Q, BN, 16), lambda i: (0, i, 0)),
            pl.BlockSpec((Q, BN, 16), lambda i: (0, i, 0)),
            pl.BlockSpec((BN, 1), lambda i: (i, 0)),
            pl.BlockSpec((1, 64), lambda i: (0, 0)),
            pl.BlockSpec((64, 64), lambda i: (0, 0)),
        ],
        out_specs=pl.BlockSpec((Q, BN, 16), lambda i: (0, i, 0)),
        out_shape=jax.ShapeDtypeStruct((Q, N, 16), jnp.float32),
    )(S, y, dinv_col, b_row, Wn)


def _tc_final(S, y, dinv_col, b_row, Wl, bl2, batch_col):
    def body(s_ref, y_ref, d_ref, b_ref, wl_ref, bl_ref, bat_ref, o_ref, acc):
        i = pl.program_id(0)
        sf = jnp.concatenate([s_ref[q] for q in range(Q)], axis=1)
        yf = jnp.concatenate([y_ref[q] for q in range(Q)], axis=1)
        e = d_ref[...] * (sf + yf) + b_ref[...]
        v = jnp.dot(e, wl_ref[...], preferred_element_type=jnp.float32)
        onehot = (bat_ref[...] == lax.broadcasted_iota(
            jnp.int32, (BN, G), 1)).astype(jnp.float32)
        vc = jnp.concatenate([v, jnp.ones((BN, 1), jnp.float32)], axis=1)
        part = lax.dot_general(vc, onehot, (((0,), (0,)), ((), ())),
                               preferred_element_type=jnp.float32)

        @pl.when(i == 0)
        def _():
            acc[...] = jnp.zeros((2, G), jnp.float32)

        acc[...] += part

        @pl.when(i == NB - 1)
        def _():
            res = acc[0, :] / jnp.maximum(acc[1, :], 1.0) + bl_ref[0, 0]
            o_ref[...] = res[None, :]

    return pl.pallas_call(
        body,
        grid=(NB,),
        in_specs=[
            pl.BlockSpec((Q, BN, 16), lambda i: (0, i, 0)),
            pl.BlockSpec((Q, BN, 16), lambda i: (0, i, 0)),
            pl.BlockSpec((BN, 1), lambda i: (i, 0)),
            pl.BlockSpec((1, 64), lambda i: (0, 0)),
            pl.BlockSpec((64, 1), lambda i: (0, 0)),
            pl.BlockSpec((1, 1), lambda i: (0, 0)),
            pl.BlockSpec((BN, 1), lambda i: (i, 0)),
        ],
        out_specs=pl.BlockSpec((1, G), lambda i: (0, 0)),
        out_shape=jax.ShapeDtypeStruct((1, G), jnp.float32),
        scratch_shapes=[pltpu.VMEM((2, G), jnp.float32)],
    )(S, y, dinv_col, b_row, Wl, bl2, batch_col)


def kernel(x, edge_index, batch, W0, b0, W1, b1, W2, b2, Wl, bl):
    src = edge_index[0]
    dst = edge_index[1]
    pad = EP - E
    src_p = jnp.concatenate([src, jnp.zeros((pad,), jnp.int32)])
    dst_p = jnp.concatenate([dst, jnp.full((pad,), DUMP, jnp.int32)])
    srcq = (src_p[None, :]
            + (jnp.arange(Q, dtype=jnp.int32) * N)[:, None]).reshape(
                Q, EP // 128, 128)
    dst2d = dst_p.reshape(EP // 128, 128)
    comb = jnp.stack(
        [srcq, jnp.broadcast_to(dst2d[None], (Q, EP // 128, 128))], axis=2)

    xp = jnp.pad(x, ((0, 0), (0, 32 - x.shape[1])))
    W0p = jnp.pad(W0, ((0, 32 - W0.shape[0]), (0, 0)))

    parts = _sc_deg(dst2d)
    dinv_col = _tc_dinv(parts).reshape(N, 1)
    batch_col = batch.reshape(N, 1)

    y = _tc_first(xp, W0p, dinv_col)
    S = _sc_scatter(y.reshape(Q * N, 16), comb).reshape(Q, ND, 16)
    y = _tc_mid(S, y, dinv_col, b0.reshape(1, H), W1)
    S = _sc_scatter(y.reshape(Q * N, 16), comb).reshape(Q, ND, 16)
    y = _tc_mid(S, y, dinv_col, b1.reshape(1, H), W2)
    S = _sc_scatter(y.reshape(Q * N, 16), comb).reshape(Q, ND, 16)
    out = _tc_final(S, y, dinv_col, b2.reshape(1, H), Wl,
                    bl.reshape(1, 1), batch_col)
    return out.reshape(G, 1)
